# baseline (speedup 1.0000x reference)
import functools

import jax
import jax.numpy as jnp
from jax import lax
from jax.experimental import pallas as pl
from jax.experimental.pallas import tpu as pltpu
from jax.experimental.pallas import tpu_sc as plsc


def _sc_neighbor_gather(s_flat, e_pad, n_states, per_tile):
    total_pad = e_pad.shape[0]
    mesh = plsc.VectorSubcoreMesh(core_axis_name="c", subcore_axis_name="s")

    @functools.partial(
        pl.kernel,
        mesh=mesh,
        out_type=jax.ShapeDtypeStruct((total_pad,), jnp.int32),
        compiler_params=pltpu.CompilerParams(needs_layout_passes=False),
        scratch_types=[
            pltpu.VMEM((n_states,), jnp.int32),
            pltpu.VMEM((per_tile,), jnp.int32),
            pltpu.VMEM((per_tile,), jnp.int32),
        ],
    )
    def k(s_hbm, e_hbm, out_hbm, s_v, e_v, o_v):
        wid = lax.axis_index("s") * 2 + lax.axis_index("c")
        base = wid * per_tile
        pltpu.sync_copy(s_hbm, s_v)
        pltpu.sync_copy(e_hbm.at[pl.ds(base, per_tile)], e_v)

        @plsc.parallel_loop(0, per_tile // 16, unroll=8)
        def body(j):
            idx = e_v[pl.ds(j * 16, 16)]
            o_v[pl.ds(j * 16, 16)] = plsc.load_gather(s_v, [idx])
        pltpu.sync_copy(o_v, out_hbm.at[pl.ds(base, per_tile)])

    return k(s_flat, e_pad)


def _potts_body(C, K, N, bn, j_ref, sj_ref, mij_ref, h_ref, mi_ref, s_ref,
                ui_ref, u_ref):
    i = pl.program_id(0)

    sjb = sj_ref[...]
    mijb = mij_ref[...]
    siota = lax.broadcasted_iota(jnp.int32, (C, K, bn), 0)
    W = jnp.where(siota == sjb[None], mijb[None], 0.0)

    rows = [jnp.sum(j_ref[0, c] * W, axis=(0, 1)) for c in range(C)]
    Ji = jnp.stack(rows, axis=0)

    mi = mi_ref[0:1, :]
    Ui = h_ref[...] * mi + Ji
    ui_ref[...] = Ui

    srow = s_ref[0:1, :]
    ciota = lax.broadcasted_iota(jnp.int32, (C, bn), 0)
    niota = lax.broadcasted_iota(jnp.int32, (1, bn), 1) + i * bn
    pick = (ciota == srow) & (niota < N)
    contrib = jnp.sum(jnp.where(pick, Ui - 0.5 * Ji, 0.0))

    @pl.when(i == 0)
    def _():
        u_ref[...] = jnp.zeros_like(u_ref)

    u_ref[...] += contrib


def _potts_call(Jt, sj_t, mij_t, ht, mi8, s8, *, C, K, N, bn,
                interpret=False):
    grid = (pl.cdiv(N, bn),)
    body = functools.partial(_potts_body, C, K, N, bn)
    return pl.pallas_call(
        body,
        grid=grid,
        in_specs=[
            pl.BlockSpec((1, C, C, K, bn), lambda i: (0, 0, 0, 0, i)),
            pl.BlockSpec((K, bn), lambda i: (0, i)),
            pl.BlockSpec((K, bn), lambda i: (0, i)),
            pl.BlockSpec((C, bn), lambda i: (0, i)),
            pl.BlockSpec((8, bn), lambda i: (0, i)),
            pl.BlockSpec((8, bn), lambda i: (0, i)),
        ],
        out_specs=[
            pl.BlockSpec((C, bn), lambda i: (0, i)),
            pl.BlockSpec((1, 1), lambda i: (0, 0)),
        ],
        out_shape=[
            jax.ShapeDtypeStruct((C, N), jnp.float32),
            jax.ShapeDtypeStruct((1, 1), jnp.float32),
        ],
        interpret=interpret,
        compiler_params=pltpu.CompilerParams(vmem_limit_bytes=120 * 2**20),
    )(Jt, sj_t, mij_t, ht, mi8, s8)


def kernel(S, h, J, edge_idx, mask_i, mask_ij):
    B, N, K = edge_idx.shape
    C = h.shape[-1]
    total = N * K

    edge_t = jnp.transpose(edge_idx, (0, 2, 1)).reshape(total)
    mij_t = jnp.transpose(mask_ij, (0, 2, 1)).reshape(K, N).astype(jnp.float32)
    Jt = jnp.transpose(J, (0, 3, 4, 2, 1))
    ht = jnp.transpose(h, (0, 2, 1)).reshape(C, N)

    n_workers = 32
    per_tile = -(-total // (n_workers * 16)) * 16
    total_pad = per_tile * n_workers
    s_flat = S.reshape(B * N).astype(jnp.int32)
    e_pad = jnp.concatenate(
        [edge_t.astype(jnp.int32), jnp.zeros((total_pad - total,), jnp.int32)])
    sj_t = _sc_neighbor_gather(s_flat, e_pad, B * N, per_tile)[:total]
    sj_t = sj_t.reshape(K, N)

    mi8 = jnp.broadcast_to(mask_i.reshape(1, N).astype(jnp.float32), (8, N))
    s8 = jnp.broadcast_to(S.reshape(1, N).astype(jnp.int32), (8, N))

    bn = 1024
    Uit, Usum = _potts_call(Jt, sj_t, mij_t, ht, mi8, s8,
                            C=C, K=K, N=N, bn=bn)
    Ui = jnp.transpose(Uit.reshape(1, C, N), (0, 2, 1))
    return (Usum.reshape(B), Ui)


# speedup vs baseline: 1.0494x; 1.0494x over previous
import functools

import jax
import jax.numpy as jnp
from jax import lax
from jax.experimental import pallas as pl
from jax.experimental.pallas import tpu as pltpu
from jax.experimental.pallas import tpu_sc as plsc


def _sc_neighbor_gather(s_flat, e_pad, n_states, per_tile):
    total_pad = e_pad.shape[0]
    mesh = plsc.VectorSubcoreMesh(core_axis_name="c", subcore_axis_name="s")

    @functools.partial(
        pl.kernel,
        mesh=mesh,
        out_type=jax.ShapeDtypeStruct((total_pad,), jnp.int32),
        compiler_params=pltpu.CompilerParams(needs_layout_passes=False),
        scratch_types=[
            pltpu.VMEM((n_states,), jnp.int32),
            pltpu.VMEM((per_tile,), jnp.int32),
            pltpu.VMEM((per_tile,), jnp.int32),
        ],
    )
    def k(s_hbm, e_hbm, out_hbm, s_v, e_v, o_v):
        wid = lax.axis_index("s") * 2 + lax.axis_index("c")
        base = wid * per_tile
        pltpu.sync_copy(s_hbm, s_v)
        pltpu.sync_copy(e_hbm.at[pl.ds(base, per_tile)], e_v)

        @plsc.parallel_loop(0, per_tile // 16, unroll=8)
        def body(j):
            idx = e_v[pl.ds(j * 16, 16)]
            o_v[pl.ds(j * 16, 16)] = plsc.load_gather(s_v, [idx])
        pltpu.sync_copy(o_v, out_hbm.at[pl.ds(base, per_tile)])

    return k(s_flat, e_pad)


def _potts_body(C, K, N, bn, j_ref, sj_ref, mij_ref, h_ref, mi_ref, s_ref,
                ui_ref, u_ref):
    i = pl.program_id(0)

    sjb = sj_ref[...]
    mijb = mij_ref[...]
    siota = lax.broadcasted_iota(jnp.int32, (C, K, bn), 0)
    W = jnp.where(siota == sjb[None], mijb[None], 0.0)

    CB = 4
    rows = []
    for c0 in range(0, C, CB):
        accs = [jnp.zeros((K, bn), jnp.float32) for _ in range(CB)]
        for s in range(C):
            Ws = W[s]
            for ci in range(CB):
                accs[ci] = accs[ci] + j_ref[0, c0 + ci, s] * Ws
        rows.extend(jnp.sum(a, axis=0) for a in accs)
    Ji = jnp.stack(rows, axis=0)

    mi = mi_ref[0:1, :]
    Ui = h_ref[...] * mi + Ji
    ui_ref[...] = Ui

    srow = s_ref[0:1, :]
    ciota = lax.broadcasted_iota(jnp.int32, (C, bn), 0)
    niota = lax.broadcasted_iota(jnp.int32, (1, bn), 1) + i * bn
    pick = (ciota == srow) & (niota < N)
    contrib = jnp.sum(jnp.where(pick, Ui - 0.5 * Ji, 0.0))

    @pl.when(i == 0)
    def _():
        u_ref[...] = jnp.zeros_like(u_ref)

    u_ref[...] += contrib


def _potts_call(Jt, sj_t, mij_t, ht, mi8, s8, *, C, K, N, bn,
                interpret=False):
    grid = (pl.cdiv(N, bn),)
    body = functools.partial(_potts_body, C, K, N, bn)
    return pl.pallas_call(
        body,
        grid=grid,
        in_specs=[
            pl.BlockSpec((1, C, C, K, bn), lambda i: (0, 0, 0, 0, i)),
            pl.BlockSpec((K, bn), lambda i: (0, i)),
            pl.BlockSpec((K, bn), lambda i: (0, i)),
            pl.BlockSpec((C, bn), lambda i: (0, i)),
            pl.BlockSpec((8, bn), lambda i: (0, i)),
            pl.BlockSpec((8, bn), lambda i: (0, i)),
        ],
        out_specs=[
            pl.BlockSpec((C, bn), lambda i: (0, i)),
            pl.BlockSpec((1, 1), lambda i: (0, 0)),
        ],
        out_shape=[
            jax.ShapeDtypeStruct((C, N), jnp.float32),
            jax.ShapeDtypeStruct((1, 1), jnp.float32),
        ],
        interpret=interpret,
        compiler_params=pltpu.CompilerParams(vmem_limit_bytes=120 * 2**20),
    )(Jt, sj_t, mij_t, ht, mi8, s8)


def kernel(S, h, J, edge_idx, mask_i, mask_ij):
    B, N, K = edge_idx.shape
    C = h.shape[-1]
    total = N * K

    edge_t = jnp.transpose(edge_idx, (0, 2, 1)).reshape(total)
    mij_t = jnp.transpose(mask_ij, (0, 2, 1)).reshape(K, N).astype(jnp.float32)
    Jt = jnp.transpose(J, (0, 3, 4, 2, 1))
    ht = jnp.transpose(h, (0, 2, 1)).reshape(C, N)

    n_workers = 32
    per_tile = -(-total // (n_workers * 16)) * 16
    total_pad = per_tile * n_workers
    s_flat = S.reshape(B * N).astype(jnp.int32)
    e_pad = jnp.concatenate(
        [edge_t.astype(jnp.int32), jnp.zeros((total_pad - total,), jnp.int32)])
    sj_t = _sc_neighbor_gather(s_flat, e_pad, B * N, per_tile)[:total]
    sj_t = sj_t.reshape(K, N)

    mi8 = jnp.broadcast_to(mask_i.reshape(1, N).astype(jnp.float32), (8, N))
    s8 = jnp.broadcast_to(S.reshape(1, N).astype(jnp.int32), (8, N))

    bn = 512
    Uit, Usum = _potts_call(Jt, sj_t, mij_t, ht, mi8, s8,
                            C=C, K=K, N=N, bn=bn)
    Ui = jnp.transpose(Uit.reshape(1, C, N), (0, 2, 1))
    return (Usum.reshape(B), Ui)
